# Initial kernel scaffold; baseline (speedup 1.0000x reference)
import jax
import jax.numpy as jnp
from jax.experimental import pallas as pl


def kernel(h, edge_index0, edge_index1, W_self0, W_neigh0, b0, W_self1, W_neigh1, b1):
    raise NotImplementedError("write your pallas kernel here")


# speedup vs baseline: 5.0887x; 5.0887x over previous
import functools

import jax
import jax.numpy as jnp
from jax import lax
from jax.experimental import pallas as pl
from jax.experimental.pallas import tpu as pltpu
from jax.experimental.pallas import tpu_sc as plsc

N = 10000
E = 320000
D = 128
N_PAD = 10240

_info = plsc.get_sparse_core_info()
NC = _info.num_cores
NS = _info.num_subcores
NW = NC * NS
EPW = E // NW
BLK = 80
NBLK = EPW // BLK
ROWS_PT = N_PAD // NS
RCHUNK = 80
NCHUNK = ROWS_PT // RCHUNK


def _sc_aggregate(h_pad, src, dst):
    mesh = plsc.VectorSubcoreMesh(core_axis_name="c", subcore_axis_name="s")

    @functools.partial(
        pl.kernel,
        mesh=mesh,
        out_type=[
            jax.ShapeDtypeStruct((NC, N_PAD, D), jnp.float32),
            jax.ShapeDtypeStruct((NC, N_PAD), jnp.float32),
        ],
        scratch_types=[
            pltpu.VMEM((BLK,), jnp.int32),
            pltpu.VMEM((BLK,), jnp.int32),
            pltpu.VMEM((RCHUNK, D), jnp.float32),
            pltpu.VMEM((BLK,), jnp.float32),
            pltpu.VMEM((ROWS_PT,), jnp.float32),
            pltpu.VMEM_SHARED((N_PAD, D), jnp.float32),
            pltpu.VMEM_SHARED((N_PAD,), jnp.float32),
            pltpu.SemaphoreType.DMA,
        ],
    )
    def body(h_hbm, src_hbm, dst_hbm, agg_out, deg_out,
             src_v, dst_v, rows_v, ones_v, vec_v, agg_sh, deg_sh, sem):
        cid = lax.axis_index("c")
        sid = lax.axis_index("s")
        wid = sid * NC + cid
        row0 = sid * ROWS_PT

        zero16 = jnp.zeros((16,), jnp.float32)
        one16 = jnp.ones((16,), jnp.float32)

        def z_rows(i, _):
            r = i // (D // 16)
            c = (i % (D // 16)) * 16
            rows_v[r, pl.ds(c, 16)] = zero16
            return 0
        lax.fori_loop(0, RCHUNK * D // 16, z_rows, 0)

        def z_vec(i, _):
            vec_v[pl.ds(i * 16, 16)] = zero16
            return 0
        lax.fori_loop(0, ROWS_PT // 16, z_vec, 0)

        def s_ones(i, _):
            ones_v[pl.ds(i * 16, 16)] = one16
            return 0
        lax.fori_loop(0, BLK // 16, s_ones, 0)

        for c in range(NCHUNK):
            pltpu.sync_copy(rows_v, agg_sh.at[pl.ds(row0 + c * RCHUNK, RCHUNK)])
        pltpu.sync_copy(vec_v, deg_sh.at[pl.ds(row0, ROWS_PT)])
        plsc.subcore_barrier()

        ebase = wid * EPW

        def edge_block(i, _):
            off = ebase + i * BLK
            pltpu.sync_copy(src_hbm.at[pl.ds(off, BLK)], src_v)
            pltpu.sync_copy(dst_hbm.at[pl.ds(off, BLK)], dst_v)
            pltpu.async_copy(h_hbm.at[src_v], rows_v, sem).wait()
            pltpu.sync_copy(rows_v, agg_sh.at[dst_v], add=True)
            pltpu.sync_copy(ones_v, deg_sh.at[dst_v], add=True)
            return 0
        lax.fori_loop(0, NBLK, edge_block, 0)
        plsc.subcore_barrier()

        for c in range(NCHUNK):
            r = row0 + c * RCHUNK
            pltpu.sync_copy(agg_sh.at[pl.ds(r, RCHUNK)], rows_v)
            pltpu.sync_copy(rows_v, agg_out.at[cid, pl.ds(r, RCHUNK)])
        pltpu.sync_copy(deg_sh.at[pl.ds(row0, ROWS_PT)], vec_v)
        pltpu.sync_copy(vec_v, deg_out.at[cid, pl.ds(row0, ROWS_PT)])

    return body(h_pad, src, dst)


def _dense_body(relu, h_ref, agg_ref, deg_ref, ws_ref, wn_ref, b_ref, o_ref):
    hv = h_ref[...]
    a = agg_ref[0] + agg_ref[1]
    dg = deg_ref[0] + deg_ref[1]
    r = 1.0 / jnp.maximum(dg, 1.0)
    hn = a * r[:, None]
    o = (jnp.dot(hv, ws_ref[...], preferred_element_type=jnp.float32)
         + jnp.dot(hn, wn_ref[...], preferred_element_type=jnp.float32)
         + b_ref[...])
    if relu:
        o = jnp.maximum(o, 0.0)
    o_ref[...] = o


def _dense_layer(h_pad, agg_part, deg_part, w_self, w_neigh, b, relu):
    BN = 256
    grid = (N_PAD // BN,)
    return pl.pallas_call(
        functools.partial(_dense_body, relu),
        grid=grid,
        in_specs=[
            pl.BlockSpec((BN, D), lambda i: (i, 0)),
            pl.BlockSpec((NC, BN, D), lambda i: (0, i, 0)),
            pl.BlockSpec((NC, BN), lambda i: (0, i)),
            pl.BlockSpec((D, D), lambda i: (0, 0)),
            pl.BlockSpec((D, D), lambda i: (0, 0)),
            pl.BlockSpec((1, D), lambda i: (0, 0)),
        ],
        out_specs=pl.BlockSpec((BN, D), lambda i: (i, 0)),
        out_shape=jax.ShapeDtypeStruct((N_PAD, D), jnp.float32),
        compiler_params=pltpu.CompilerParams(
            dimension_semantics=("arbitrary",),
        ),
    )(h_pad, agg_part, deg_part, w_self, w_neigh, b.reshape(1, D))


def kernel(h, edge_index0, edge_index1, W_self0, W_neigh0, b0,
           W_self1, W_neigh1, b1):
    src0 = edge_index0[0].astype(jnp.int32)
    dst0 = edge_index0[1].astype(jnp.int32)
    src1 = edge_index1[0].astype(jnp.int32)
    dst1 = edge_index1[1].astype(jnp.int32)
    h_pad = jnp.pad(h, ((0, N_PAD - N), (0, 0)))

    agg0, deg0 = _sc_aggregate(h_pad, src0, dst0)
    x = _dense_layer(h_pad, agg0, deg0, W_self0, W_neigh0, b0, relu=True)
    agg1, deg1 = _sc_aggregate(x, src1, dst1)
    out = _dense_layer(x, agg1, deg1, W_self1, W_neigh1, b1, relu=False)
    return out[:N]


# speedup vs baseline: 10.7074x; 2.1041x over previous
import functools

import jax
import jax.numpy as jnp
from jax import lax
from jax.experimental import pallas as pl
from jax.experimental.pallas import tpu as pltpu
from jax.experimental.pallas import tpu_sc as plsc

N = 10000
E = 320000
D = 128
N_PAD = 10240

_info = plsc.get_sparse_core_info()
NC = _info.num_cores
NS = _info.num_subcores
NW = NC * NS
EPW = E // NW
BLK = 80
NBLK = EPW // BLK
GRP = 25
NGRP = NBLK // GRP
ROWS_PT = N_PAD // NS
RCHUNK = 80
NCHUNK = ROWS_PT // RCHUNK


def _sc_aggregate(h_pad, src3, dst3):
    mesh = plsc.VectorSubcoreMesh(core_axis_name="c", subcore_axis_name="s")

    @functools.partial(
        pl.kernel,
        mesh=mesh,
        out_type=[
            jax.ShapeDtypeStruct((NC, N_PAD, D), jnp.float32),
            jax.ShapeDtypeStruct((NC, N_PAD), jnp.float32),
        ],
        scratch_types=[
            pltpu.VMEM((2, GRP, BLK), jnp.int32),
            pltpu.VMEM((2, GRP, BLK), jnp.int32),
            pltpu.VMEM((BLK, D), jnp.float32),
            pltpu.VMEM((BLK, D), jnp.float32),
            pltpu.VMEM((BLK,), jnp.float32),
            pltpu.VMEM((ROWS_PT,), jnp.float32),
            pltpu.VMEM_SHARED((N_PAD, D), jnp.float32),
            pltpu.VMEM_SHARED((N_PAD,), jnp.float32),
            pltpu.SemaphoreType.DMA,
            pltpu.SemaphoreType.DMA,
            pltpu.SemaphoreType.DMA,
            pltpu.SemaphoreType.DMA,
            pltpu.SemaphoreType.DMA,
        ],
    )
    def body(h_hbm, src_hbm, dst_hbm, agg_out, deg_out,
             src_v, dst_v, buf_a, buf_b, ones_v, vec_v, agg_sh, deg_sh,
             sem_a, sem_b, sem_d, sem_i0, sem_i1):
        cid = lax.axis_index("c")
        sid = lax.axis_index("s")
        wid = sid * NC + cid
        row0 = sid * ROWS_PT
        isems = (sem_i0, sem_i1)

        def start_refill(g):
            p = g % 2
            pltpu.async_copy(src_hbm.at[wid, g], src_v.at[p], isems[p])
            pltpu.async_copy(dst_hbm.at[wid, g], dst_v.at[p], isems[p])

        def wait_refill(g):
            p = g % 2
            pltpu.make_async_copy(src_hbm.at[wid, 0], src_v.at[p],
                                  isems[p]).wait()
            pltpu.make_async_copy(dst_hbm.at[wid, 0], dst_v.at[p],
                                  isems[p]).wait()

        start_refill(0)

        zero16 = jnp.zeros((16,), jnp.float32)
        one16 = jnp.ones((16,), jnp.float32)

        def z_rows(i, _):
            r = i // (D // 16)
            c = (i % (D // 16)) * 16
            buf_a[r, pl.ds(c, 16)] = zero16
            return 0
        lax.fori_loop(0, BLK * D // 16, z_rows, 0)

        def z_vec(i, _):
            vec_v[pl.ds(i * 16, 16)] = zero16
            return 0
        lax.fori_loop(0, ROWS_PT // 16, z_vec, 0)

        def s_ones(i, _):
            ones_v[pl.ds(i * 16, 16)] = one16
            return 0
        lax.fori_loop(0, BLK // 16, s_ones, 0)

        for c in range(NCHUNK):
            pltpu.sync_copy(buf_a, agg_sh.at[pl.ds(row0 + c * RCHUNK, RCHUNK)])
        pltpu.sync_copy(vec_v, deg_sh.at[pl.ds(row0, ROWS_PT)])
        plsc.subcore_barrier()

        def start_gather(p, k, buf, sem):
            pltpu.async_copy(h_hbm.at[src_v.at[p, k]], buf, sem)

        def wait_gather(buf, sem):
            pltpu.make_async_copy(h_hbm.at[src_v.at[0, 0]], buf, sem).wait()

        def start_deg(p, k):
            pltpu.async_copy(ones_v, deg_sh.at[dst_v.at[p, k]], sem_d,
                             add=True)

        def wait_deg(p, k):
            pltpu.make_async_copy(ones_v, deg_sh.at[dst_v.at[p, k]],
                                  sem_d).wait()

        for g in range(NGRP):
            p = g % 2
            wait_refill(g)
            if g + 1 < NGRP:
                start_refill(g + 1)

            start_gather(p, 0, buf_a, sem_a)
            start_gather(p, 1, buf_b, sem_b)
            start_deg(p, 0)

            def half_step(k, buf, sem):
                wait_gather(buf, sem)
                pltpu.sync_copy(buf, agg_sh.at[dst_v.at[p, k]], add=True)
                wait_deg(p, k)
                start_deg(p, jnp.minimum(k + 1, GRP - 1))
                start_gather(p, jnp.minimum(k + 2, GRP - 1), buf, sem)

            def double_step(j, _):
                half_step(2 * j, buf_a, sem_a)
                half_step(2 * j + 1, buf_b, sem_b)
                return 0
            lax.fori_loop(0, (GRP - 1) // 2, double_step, 0)

            wait_gather(buf_a, sem_a)
            pltpu.sync_copy(buf_a, agg_sh.at[dst_v.at[p, GRP - 1]], add=True)
            wait_deg(p, GRP - 1)
            wait_gather(buf_b, sem_b)
        plsc.subcore_barrier()

        for c in range(NCHUNK):
            r = row0 + c * RCHUNK
            pltpu.sync_copy(agg_sh.at[pl.ds(r, RCHUNK)], buf_a)
            pltpu.sync_copy(buf_a, agg_out.at[cid, pl.ds(r, RCHUNK)])
        pltpu.sync_copy(deg_sh.at[pl.ds(row0, ROWS_PT)], vec_v)
        pltpu.sync_copy(vec_v, deg_out.at[cid, pl.ds(row0, ROWS_PT)])

    return body(h_pad, src3, dst3)


def _dense_body(relu, h_ref, agg_ref, deg_ref, ws_ref, wn_ref, b_ref, o_ref):
    hv = h_ref[...]
    a = agg_ref[0] + agg_ref[1]
    dg = deg_ref[0] + deg_ref[1]
    r = 1.0 / jnp.maximum(dg, 1.0)
    hn = a * r[:, None]
    o = (jnp.dot(hv, ws_ref[...], preferred_element_type=jnp.float32)
         + jnp.dot(hn, wn_ref[...], preferred_element_type=jnp.float32)
         + b_ref[...])
    if relu:
        o = jnp.maximum(o, 0.0)
    o_ref[...] = o


def _dense_layer(h_pad, agg_part, deg_part, w_self, w_neigh, b, relu):
    BN = 256
    grid = (N_PAD // BN,)
    return pl.pallas_call(
        functools.partial(_dense_body, relu),
        grid=grid,
        in_specs=[
            pl.BlockSpec((BN, D), lambda i: (i, 0)),
            pl.BlockSpec((NC, BN, D), lambda i: (0, i, 0)),
            pl.BlockSpec((NC, BN), lambda i: (0, i)),
            pl.BlockSpec((D, D), lambda i: (0, 0)),
            pl.BlockSpec((D, D), lambda i: (0, 0)),
            pl.BlockSpec((1, D), lambda i: (0, 0)),
        ],
        out_specs=pl.BlockSpec((BN, D), lambda i: (i, 0)),
        out_shape=jax.ShapeDtypeStruct((N_PAD, D), jnp.float32),
        compiler_params=pltpu.CompilerParams(
            dimension_semantics=("arbitrary",),
        ),
    )(h_pad, agg_part, deg_part, w_self, w_neigh, b.reshape(1, D))


def kernel(h, edge_index0, edge_index1, W_self0, W_neigh0, b0,
           W_self1, W_neigh1, b1):
    src0 = edge_index0[0].astype(jnp.int32).reshape(NW, NGRP, GRP, BLK)
    dst0 = edge_index0[1].astype(jnp.int32).reshape(NW, NGRP, GRP, BLK)
    src1 = edge_index1[0].astype(jnp.int32).reshape(NW, NGRP, GRP, BLK)
    dst1 = edge_index1[1].astype(jnp.int32).reshape(NW, NGRP, GRP, BLK)
    h_pad = jnp.pad(h, ((0, N_PAD - N), (0, 0)))

    agg0, deg0 = _sc_aggregate(h_pad, src0, dst0)
    x = _dense_layer(h_pad, agg0, deg0, W_self0, W_neigh0, b0, relu=True)
    agg1, deg1 = _sc_aggregate(x, src1, dst1)
    out = _dense_layer(x, agg1, deg1, W_self1, W_neigh1, b1, relu=False)
    return out[:N]
